# baseline (speedup 1.0000x reference)
import functools

import jax
import jax.numpy as jnp
from jax import lax
from jax.experimental import pallas as pl
from jax.experimental.pallas import tpu as pltpu
from jax.experimental.pallas import tpu_sc as plsc

V = 100000
D = 70
B = 1024
L = 50

NC = 2
NS = 16
NW = NC * NS
BPW = B // NW
GROUP = 4
NG = BPW // GROUP
ROWS_PER_G = GROUP * L
DP = 72

CHUNK_OFFS = (0, 16, 32, 48, 54)


def _make_pool_kernel():
    mesh = plsc.VectorSubcoreMesh(core_axis_name="c", subcore_axis_name="s")

    @functools.partial(
        pl.kernel,
        mesh=mesh,
        out_type=jax.ShapeDtypeStruct((B, DP), jnp.float32),
        scratch_types=[
            pltpu.VMEM((NG, ROWS_PER_G), jnp.int32),
            pltpu.VMEM((NG, ROWS_PER_G, DP), jnp.float32),
            pltpu.VMEM((BPW, DP), jnp.float32),
            pltpu.SemaphoreType.DMA,
        ],
        compiler_params=pltpu.CompilerParams(use_tc_tiling_on_sc=False),
    )
    def pool(x_hbm, table_hbm, out_hbm, idx_v, buf, stage, sem):
        wid = lax.axis_index("s") * NC + lax.axis_index("c")
        pltpu.sync_copy(x_hbm.at[pl.ds(wid * NG, NG)], idx_v)
        copies = []
        for g in range(NG):
            copies.append(
                pltpu.async_copy(table_hbm.at[idx_v.at[g]], buf.at[g], sem)
            )
        zero = jnp.zeros((16,), jnp.float32)
        for g in range(NG):
            copies[g].wait()
            for j in range(GROUP):
                def body(i, accs, g=g, j=j):
                    r = j * L + i
                    return tuple(
                        acc + buf[g, r, pl.ds(off, 16)]
                        for acc, off in zip(accs, CHUNK_OFFS)
                    )
                accs = lax.fori_loop(0, L, body, (zero,) * 5)
                row = g * GROUP + j
                for acc, off in zip(accs, CHUNK_OFFS):
                    stage[row, pl.ds(off, 16)] = acc
        pltpu.sync_copy(stage, out_hbm.at[pl.ds(wid * BPW, BPW)])

    return pool


_pool = _make_pool_kernel()

VT = 2048


def _mm_body(p_ref, w_ref, b_ref, o_ref):
    p = p_ref[...][:, :D] * (1.0 / L)
    w = w_ref[...]
    acc = lax.dot_general(
        p, w, (((1,), (1,)), ((), ())), preferred_element_type=jnp.float32
    )
    o_ref[...] = acc + b_ref[...]


def _projection(pooled, W, b2):
    grid = (pl.cdiv(V, VT),)
    return pl.pallas_call(
        _mm_body,
        grid=grid,
        in_specs=[
            pl.BlockSpec((B, DP), lambda i: (0, 0)),
            pl.BlockSpec((VT, D), lambda i: (i, 0)),
            pl.BlockSpec((1, VT), lambda i: (0, i)),
        ],
        out_specs=pl.BlockSpec((B, VT), lambda i: (0, i)),
        out_shape=jax.ShapeDtypeStruct((B, V), jnp.float32),
        compiler_params=pltpu.CompilerParams(
            dimension_semantics=("parallel",),
        ),
    )(pooled, W, b2)


def kernel(x, table, W, b):
    xi = x.astype(jnp.int32).reshape(NW * NG, ROWS_PER_G)
    table_p = jnp.pad(table, ((0, 0), (0, DP - D)))
    pooled = _pool(xi, table_p)
    return _projection(pooled, W, b.reshape(1, V))
